# baseline (speedup 1.0000x reference)
import functools

import jax
import jax.numpy as jnp
from jax import lax
from jax.experimental import pallas as pl
from jax.experimental.pallas import tpu as pltpu
from jax.experimental.pallas import tpu_sc as plsc

_N = 10000
_D = 128
_G = 12
_L = 3
_EPS = 1e-5
_NP = 10240
_NW = 32
_CH = 128
_CPW = 80
_EPW = _CH * _CPW
_EP = _NW * _EPW
_RPT = _NP // 16
_DEGW = 128
_BN = 1024
_GRID = _NP // _BN

_f32 = jnp.float32
_mesh = plsc.VectorSubcoreMesh(core_axis_name="c", subcore_axis_name="s")


@functools.partial(
    pl.kernel,
    out_type=jax.ShapeDtypeStruct((2, _NP, _DEGW), _f32),
    mesh=_mesh,
    scratch_types=[
        pltpu.VMEM((_CPW, _CH), jnp.int32),
        pltpu.VMEM((_CH, _DEGW), _f32),
        pltpu.VMEM((_CH, _DEGW), _f32),
        pltpu.VMEM_SHARED((_NP, _DEGW), _f32),
    ],
)
def _deg_kernel(dst_hbm, out_hbm, didx, zer, one, acc):
    c = lax.axis_index("c")
    s = lax.axis_index("s")
    wid = s * 2 + c

    def fill(i, carry):
        for j in range(_DEGW // 16):
            zer[i, pl.ds(j * 16, 16)] = jnp.zeros((16,), _f32)
            one[i, pl.ds(j * 16, 16)] = jnp.ones((16,), _f32)
        return carry

    lax.fori_loop(0, _CH, fill, 0)
    base = s * _RPT
    for k in range(_RPT // _CH):
        pltpu.sync_copy(zer, acc.at[pl.ds(base + k * _CH, _CH)])
    plsc.subcore_barrier()

    pltpu.sync_copy(dst_hbm.at[pl.ds(wid * _CPW, _CPW)], didx)

    def step(k, carry):
        pltpu.sync_copy(one, acc.at[didx.at[k]], add=True)
        return carry

    lax.fori_loop(0, _CPW, step, 0)
    plsc.subcore_barrier()
    pltpu.sync_copy(acc.at[pl.ds(base, _RPT)], out_hbm.at[c, pl.ds(base, _RPT)])


_NBUF = 2
_LA = 1
_TCH = _EP // _CH
_CPS = _TCH // 16
_A0 = 160
_A1 = _CPS - _A0
_IDXR = 32
_NPH0 = _A0 // _IDXR
_NPH1 = _A1 // _IDXR


@functools.partial(
    pl.kernel,
    out_type=jax.ShapeDtypeStruct((2, _NP, _D), _f32),
    mesh=_mesh,
    scratch_types=[
        pltpu.VMEM((_IDXR, _CH), jnp.int32),
        pltpu.VMEM((_IDXR, _CH), jnp.int32),
    ] + [pltpu.VMEM((_CH, _D), _f32) for _ in range(_NBUF)]
      + [pltpu.SemaphoreType.DMA for _ in range(2 * _NBUF)]
      + [pltpu.VMEM_SHARED((_NP, _D), _f32)],
)
def _agg_kernel(hp_hbm, src_hbm, dst_hbm, out_hbm, sidx, didx, *rest):
    rows = rest[:_NBUF]
    gsem = rest[_NBUF:2 * _NBUF]
    ssem = rest[2 * _NBUF:3 * _NBUF]
    acc = rest[3 * _NBUF]
    c = lax.axis_index("c")
    s = lax.axis_index("s")

    def fill(i, carry):
        for j in range(_D // 16):
            rows[0][i, pl.ds(j * 16, 16)] = jnp.zeros((16,), _f32)
        return carry

    lax.fori_loop(0, _CH, fill, 0)
    base = s * _RPT
    for k in range(_RPT // _CH):
        pltpu.sync_copy(rows[0], acc.at[pl.ds(base + k * _CH, _CH)])
    plsc.subcore_barrier()

    woff = s * _CPS + c * _A0

    def gather(k, b):
        return pltpu.make_async_copy(hp_hbm.at[sidx.at[k]], rows[b], gsem[b])

    def scatter(k, b):
        return pltpu.make_async_copy(rows[b], acc.at[didx.at[k]], ssem[b])

    def stage(g, b, prologue, epilogue):
        k = g * _NBUF + b
        gather(k, b).wait()
        scatter(k, b).start(add=True)
        t = k + _LA
        b2 = (b + _LA) % _NBUF
        if not epilogue:
            if not (prologue and b < _NBUF - _LA):
                scatter(t - _NBUF, b2).wait()
            gather(t, b2).start()

    def run(nph):
        pch = _IDXR
        for ph in range(nph):
            poff = woff + ph * pch
            pltpu.sync_copy(src_hbm.at[pl.ds(poff, _IDXR)], sidx)
            pltpu.sync_copy(dst_hbm.at[pl.ds(poff, _IDXR)], didx)

            for b in range(_NBUF):
                if b < _LA:
                    gather(b, b).start()
                stage(0, b, True, False)

            def group(g, carry):
                for b in range(_NBUF):
                    stage(g, b, False, False)
                return carry

            lax.fori_loop(1, pch // _NBUF - 1, group, 0)

            for b in range(_NBUF):
                g = pch // _NBUF - 1
                if b < _NBUF - _LA:
                    stage(g, b, False, False)
                else:
                    stage(g, b, False, True)
            for b in range(_NBUF):
                scatter(pch - _NBUF + b, b).wait()

    @pl.when(c == 0)
    def _():
        run(_NPH0)

    @pl.when(c == 1)
    def _():
        run(_NPH1)

    plsc.subcore_barrier()
    pltpu.sync_copy(acc.at[pl.ds(base, _RPT)], out_hbm.at[c, pl.ds(base, _RPT)])


def _prep_body(deg_ref, x_ref, w_ref, dinv_ref, hp_ref):
    deg = deg_ref[0, :, 0:1] + deg_ref[1, :, 0:1] + 1.0
    dinv = lax.rsqrt(deg)
    dinv_ref[...] = dinv
    hp_ref[...] = jnp.dot(x_ref[...], w_ref[...],
                          preferred_element_type=_f32) * dinv


_prep = pl.pallas_call(
    _prep_body,
    grid=(_GRID,),
    in_specs=[
        pl.BlockSpec((2, _BN, _DEGW), lambda i: (0, i, 0)),
        pl.BlockSpec((_BN, _D), lambda i: (i, 0)),
        pl.BlockSpec((_D, _D), lambda i: (0, 0)),
    ],
    out_specs=[
        pl.BlockSpec((_BN, 1), lambda i: (i, 0)),
        pl.BlockSpec((_BN, _D), lambda i: (i, 0)),
    ],
    out_shape=[
        jax.ShapeDtypeStruct((_NP, 1), _f32),
        jax.ShapeDtypeStruct((_NP, _D), _f32),
    ],
)


def _norm_block(z, lng, lnb):
    m = jnp.mean(z, axis=-1, keepdims=True)
    zc = z - m
    v = jnp.mean(zc * zc, axis=-1, keepdims=True)
    return zc * lax.rsqrt(v + _EPS) * lng + lnb


def _mid_body(agg_ref, hp_ref, h_ref, dinv_ref, bg_ref, lng_ref, lnb_ref,
              w_ref, hn_ref, hpn_ref):
    dinv = dinv_ref[...]
    z = dinv * (agg_ref[0] + agg_ref[1] + hp_ref[...]) + bg_ref[...]
    zn = _norm_block(z, lng_ref[...], lnb_ref[...])
    hn = jnp.maximum(zn, 0.0) + h_ref[...]
    hn_ref[...] = hn
    hpn_ref[...] = jnp.dot(hn, w_ref[...],
                           preferred_element_type=_f32) * dinv


_mid = pl.pallas_call(
    _mid_body,
    grid=(_GRID,),
    in_specs=[
        pl.BlockSpec((2, _BN, _D), lambda i: (0, i, 0)),
        pl.BlockSpec((_BN, _D), lambda i: (i, 0)),
        pl.BlockSpec((_BN, _D), lambda i: (i, 0)),
        pl.BlockSpec((_BN, 1), lambda i: (i, 0)),
        pl.BlockSpec((1, _D), lambda i: (0, 0)),
        pl.BlockSpec((1, _D), lambda i: (0, 0)),
        pl.BlockSpec((1, _D), lambda i: (0, 0)),
        pl.BlockSpec((_D, _D), lambda i: (0, 0)),
    ],
    out_specs=[
        pl.BlockSpec((_BN, _D), lambda i: (i, 0)),
        pl.BlockSpec((_BN, _D), lambda i: (i, 0)),
    ],
    out_shape=[
        jax.ShapeDtypeStruct((_NP, _D), _f32),
        jax.ShapeDtypeStruct((_NP, _D), _f32),
    ],
)


def _fin_body(agg_ref, hp_ref, h_ref, dinv_ref, bg_ref, lng_ref, lnb_ref,
              go_ref, bo_ref, piw_ref, pib_ref, muw_ref, mub_ref,
              sgw_ref, sgb_ref, ms_ref, pi_ref, mu_ref, sg_ref):
    dinv = dinv_ref[...]
    z = dinv * (agg_ref[0] + agg_ref[1] + hp_ref[...]) + bg_ref[...]
    zn = _norm_block(z, lng_ref[...], lnb_ref[...])
    h3 = jnp.maximum(zn, 0.0) + h_ref[...]
    g = _norm_block(h3, go_ref[...], bo_ref[...])
    lg = jnp.dot(g, piw_ref[...], preferred_element_type=_f32) + pib_ref[...]
    mx = jnp.max(lg, axis=-1, keepdims=True)
    e = jnp.exp(lg - mx)
    pi_ref[...] = e / jnp.sum(e, axis=-1, keepdims=True)
    mu_ref[...] = jnp.dot(g, muw_ref[...],
                          preferred_element_type=_f32) + mub_ref[...]
    sg_ref[...] = jnp.exp(jnp.dot(g, sgw_ref[...],
                                  preferred_element_type=_f32)
                          + sgb_ref[...]) + ms_ref[0, 0]


_fin = pl.pallas_call(
    _fin_body,
    grid=(_GRID,),
    in_specs=[
        pl.BlockSpec((2, _BN, _D), lambda i: (0, i, 0)),
        pl.BlockSpec((_BN, _D), lambda i: (i, 0)),
        pl.BlockSpec((_BN, _D), lambda i: (i, 0)),
        pl.BlockSpec((_BN, 1), lambda i: (i, 0)),
        pl.BlockSpec((1, _D), lambda i: (0, 0)),
        pl.BlockSpec((1, _D), lambda i: (0, 0)),
        pl.BlockSpec((1, _D), lambda i: (0, 0)),
        pl.BlockSpec((1, _D), lambda i: (0, 0)),
        pl.BlockSpec((1, _D), lambda i: (0, 0)),
        pl.BlockSpec((_D, _G), lambda i: (0, 0)),
        pl.BlockSpec((1, _G), lambda i: (0, 0)),
        pl.BlockSpec((_D, _G), lambda i: (0, 0)),
        pl.BlockSpec((1, _G), lambda i: (0, 0)),
        pl.BlockSpec((_D, _G), lambda i: (0, 0)),
        pl.BlockSpec((1, _G), lambda i: (0, 0)),
        pl.BlockSpec((1, 1), lambda i: (0, 0)),
    ],
    out_specs=[
        pl.BlockSpec((_BN, _G), lambda i: (i, 0)),
        pl.BlockSpec((_BN, _G), lambda i: (i, 0)),
        pl.BlockSpec((_BN, _G), lambda i: (i, 0)),
    ],
    out_shape=[
        jax.ShapeDtypeStruct((_NP, _G), _f32),
        jax.ShapeDtypeStruct((_NP, _G), _f32),
        jax.ShapeDtypeStruct((_NP, _G), _f32),
    ],
)


def kernel(x, edge_index, Wg, bg, lng, lnb, g_out, b_out, pi_W, pi_b,
           mu_W, mu_b, sigma_W, sigma_b, min_sigma):
    n = x.shape[0]
    e = edge_index.shape[1]
    src = edge_index[0].astype(jnp.int32)
    dst = edge_index[1].astype(jnp.int32)
    pad_e = jnp.full((_EP - e,), _NP - 1, jnp.int32)
    src2d = jnp.concatenate([src, pad_e]).reshape(_TCH, _CH)
    dst2d = jnp.concatenate([dst, pad_e]).reshape(_TCH, _CH)
    x_p = jnp.pad(x, ((0, _NP - n), (0, 0)))

    deg_parts = _deg_kernel(dst2d)
    dinv, hp = _prep(deg_parts, x_p, Wg[0])

    h = x_p
    for l in range(_L - 1):
        aggp = _agg_kernel(hp, src2d, dst2d)
        h, hp = _mid(aggp, hp, h, dinv,
                     bg[l].reshape(1, _D), lng[l].reshape(1, _D),
                     lnb[l].reshape(1, _D), Wg[l + 1])

    aggp = _agg_kernel(hp, src2d, dst2d)
    l = _L - 1
    pi, mu, sg = _fin(aggp, hp, h, dinv,
                      bg[l].reshape(1, _D), lng[l].reshape(1, _D),
                      lnb[l].reshape(1, _D),
                      g_out.reshape(1, _D), b_out.reshape(1, _D),
                      pi_W, pi_b.reshape(1, _G),
                      mu_W, mu_b.reshape(1, _G),
                      sigma_W, sigma_b.reshape(1, _G),
                      min_sigma.reshape(1, 1))

    pi = pi[:n]
    mu = mu[:n].reshape(n, _G, 1)
    sg = sg[:n].reshape(n, _G, 1)
    return (pi, mu, sg)


# speedup vs baseline: 1.6091x; 1.6091x over previous
import functools

import jax
import jax.numpy as jnp
from jax import lax
from jax.experimental import pallas as pl
from jax.experimental.pallas import tpu as pltpu
from jax.experimental.pallas import tpu_sc as plsc

_N = 10000
_D = 128
_G = 12
_L = 3
_EPS = 1e-5
_NP = 10240
_NW = 32
_CH = 128
_CPW = 80
_EPW = _CH * _CPW
_EP = _NW * _EPW
_RPT = _NP // 16
_DEGW = 128
_BN = 1024
_GRID = _NP // _BN

_f32 = jnp.float32
_mesh = plsc.VectorSubcoreMesh(core_axis_name="c", subcore_axis_name="s")


@functools.partial(
    pl.kernel,
    out_type=jax.ShapeDtypeStruct((2, _NP, _DEGW), _f32),
    mesh=_mesh,
    scratch_types=[
        pltpu.VMEM((_CPW, _CH), jnp.int32),
        pltpu.VMEM((_CH, _DEGW), _f32),
        pltpu.VMEM((_CH, _DEGW), _f32),
        pltpu.VMEM_SHARED((_NP, _DEGW), _f32),
    ],
)
def _deg_kernel(dst_hbm, out_hbm, didx, zer, one, acc):
    c = lax.axis_index("c")
    s = lax.axis_index("s")
    wid = s * 2 + c

    def fill(i, carry):
        for j in range(_DEGW // 16):
            zer[i, pl.ds(j * 16, 16)] = jnp.zeros((16,), _f32)
            one[i, pl.ds(j * 16, 16)] = jnp.ones((16,), _f32)
        return carry

    lax.fori_loop(0, _CH, fill, 0)
    base = s * _RPT
    for k in range(_RPT // _CH):
        pltpu.sync_copy(zer, acc.at[pl.ds(base + k * _CH, _CH)])
    plsc.subcore_barrier()

    pltpu.sync_copy(dst_hbm.at[pl.ds(wid * _CPW, _CPW)], didx)

    def step(k, carry):
        pltpu.sync_copy(one, acc.at[didx.at[k]], add=True)
        return carry

    lax.fori_loop(0, _CPW, step, 0)
    plsc.subcore_barrier()
    pltpu.sync_copy(acc.at[pl.ds(base, _RPT)], out_hbm.at[c, pl.ds(base, _RPT)])


_NBUF = 2
_LA = 1
_TCH = _EP // _CH
_CPS = _TCH // 16
_A0 = 144
_A1 = _CPS - _A0
_IDXR = 32
_NPH0 = _A0 // _IDXR
_NPH1 = _A1 // _IDXR


@functools.partial(
    pl.kernel,
    out_type=jax.ShapeDtypeStruct((2, _NP, _D), _f32),
    mesh=_mesh,
    scratch_types=[
        pltpu.VMEM((_IDXR, _CH), jnp.int32),
        pltpu.VMEM((_IDXR, _CH), jnp.int32),
    ] + [pltpu.VMEM((_CH, _D), _f32) for _ in range(_NBUF)]
      + [pltpu.SemaphoreType.DMA for _ in range(2 * _NBUF)]
      + [pltpu.VMEM_SHARED((_NP, _D), _f32)],
)
def _agg_kernel(hp_hbm, src_hbm, dst_hbm, out_hbm, sidx, didx, *rest):
    rows = rest[:_NBUF]
    gsem = rest[_NBUF:2 * _NBUF]
    ssem = rest[2 * _NBUF:3 * _NBUF]
    acc = rest[3 * _NBUF]
    c = lax.axis_index("c")
    s = lax.axis_index("s")

    def fill(i, carry):
        for j in range(_D // 16):
            rows[0][i, pl.ds(j * 16, 16)] = jnp.zeros((16,), _f32)
        return carry

    lax.fori_loop(0, _CH, fill, 0)
    base = s * _RPT
    for k in range(_RPT // _CH):
        pltpu.sync_copy(rows[0], acc.at[pl.ds(base + k * _CH, _CH)])
    plsc.subcore_barrier()

    woff = s * _CPS + c * _A0

    def gather(k, b):
        return pltpu.make_async_copy(hp_hbm.at[sidx.at[k]], rows[b], gsem[b])

    def scatter(k, b):
        return pltpu.make_async_copy(rows[b], acc.at[didx.at[k]], ssem[b])

    def stage(g, b, prologue, epilogue):
        k = g * _NBUF + b
        gather(k, b).wait()
        scatter(k, b).start(add=True)
        t = k + _LA
        b2 = (b + _LA) % _NBUF
        if not epilogue:
            if not (prologue and b < _NBUF - _LA):
                scatter(t - _NBUF, b2).wait()
            gather(t, b2).start()

    def run(nph):
        pch = _IDXR
        for ph in range(nph):
            poff = woff + ph * pch
            pltpu.sync_copy(src_hbm.at[pl.ds(poff, _IDXR)], sidx)
            pltpu.sync_copy(dst_hbm.at[pl.ds(poff, _IDXR)], didx)

            for b in range(_NBUF):
                if b < _LA:
                    gather(b, b).start()
                stage(0, b, True, False)

            def group(g, carry):
                for b in range(_NBUF):
                    stage(g, b, False, False)
                return carry

            lax.fori_loop(1, pch // _NBUF - 1, group, 0)

            for b in range(_NBUF):
                g = pch // _NBUF - 1
                if b < _NBUF - _LA:
                    stage(g, b, False, False)
                else:
                    stage(g, b, False, True)
            for b in range(_NBUF):
                scatter(pch - _NBUF + b, b).wait()

    @pl.when(c == 0)
    def _():
        run(_NPH0)

    @pl.when(c == 1)
    def _():
        run(_NPH1)

    plsc.subcore_barrier()
    pltpu.sync_copy(acc.at[pl.ds(base, _RPT)], out_hbm.at[c, pl.ds(base, _RPT)])


def _prep_body(deg_ref, x_ref, w_ref, dinv_ref, hp_ref):
    deg = deg_ref[0, :, 0:1] + deg_ref[1, :, 0:1] + 1.0
    dinv = lax.rsqrt(deg)
    dinv_ref[...] = dinv
    hp_ref[...] = jnp.dot(x_ref[...], w_ref[...],
                          preferred_element_type=_f32) * dinv


_prep = pl.pallas_call(
    _prep_body,
    grid=(_GRID,),
    in_specs=[
        pl.BlockSpec((2, _BN, _DEGW), lambda i: (0, i, 0)),
        pl.BlockSpec((_BN, _D), lambda i: (i, 0)),
        pl.BlockSpec((_D, _D), lambda i: (0, 0)),
    ],
    out_specs=[
        pl.BlockSpec((_BN, 1), lambda i: (i, 0)),
        pl.BlockSpec((_BN, _D), lambda i: (i, 0)),
    ],
    out_shape=[
        jax.ShapeDtypeStruct((_NP, 1), _f32),
        jax.ShapeDtypeStruct((_NP, _D), _f32),
    ],
)


def _norm_block(z, lng, lnb):
    m = jnp.mean(z, axis=-1, keepdims=True)
    zc = z - m
    v = jnp.mean(zc * zc, axis=-1, keepdims=True)
    return zc * lax.rsqrt(v + _EPS) * lng + lnb


def _mid_body(agg_ref, hp_ref, h_ref, dinv_ref, bg_ref, lng_ref, lnb_ref,
              w_ref, hn_ref, hpn_ref):
    dinv = dinv_ref[...]
    z = dinv * (agg_ref[0] + agg_ref[1] + hp_ref[...]) + bg_ref[...]
    zn = _norm_block(z, lng_ref[...], lnb_ref[...])
    hn = jnp.maximum(zn, 0.0) + h_ref[...]
    hn_ref[...] = hn
    hpn_ref[...] = jnp.dot(hn, w_ref[...],
                           preferred_element_type=_f32) * dinv


_mid = pl.pallas_call(
    _mid_body,
    grid=(_GRID,),
    in_specs=[
        pl.BlockSpec((2, _BN, _D), lambda i: (0, i, 0)),
        pl.BlockSpec((_BN, _D), lambda i: (i, 0)),
        pl.BlockSpec((_BN, _D), lambda i: (i, 0)),
        pl.BlockSpec((_BN, 1), lambda i: (i, 0)),
        pl.BlockSpec((1, _D), lambda i: (0, 0)),
        pl.BlockSpec((1, _D), lambda i: (0, 0)),
        pl.BlockSpec((1, _D), lambda i: (0, 0)),
        pl.BlockSpec((_D, _D), lambda i: (0, 0)),
    ],
    out_specs=[
        pl.BlockSpec((_BN, _D), lambda i: (i, 0)),
        pl.BlockSpec((_BN, _D), lambda i: (i, 0)),
    ],
    out_shape=[
        jax.ShapeDtypeStruct((_NP, _D), _f32),
        jax.ShapeDtypeStruct((_NP, _D), _f32),
    ],
)


def _fin_body(agg_ref, hp_ref, h_ref, dinv_ref, bg_ref, lng_ref, lnb_ref,
              go_ref, bo_ref, piw_ref, pib_ref, muw_ref, mub_ref,
              sgw_ref, sgb_ref, ms_ref, pi_ref, mu_ref, sg_ref):
    dinv = dinv_ref[...]
    z = dinv * (agg_ref[0] + agg_ref[1] + hp_ref[...]) + bg_ref[...]
    zn = _norm_block(z, lng_ref[...], lnb_ref[...])
    h3 = jnp.maximum(zn, 0.0) + h_ref[...]
    g = _norm_block(h3, go_ref[...], bo_ref[...])
    lg = jnp.dot(g, piw_ref[...], preferred_element_type=_f32) + pib_ref[...]
    mx = jnp.max(lg, axis=-1, keepdims=True)
    e = jnp.exp(lg - mx)
    pi_ref[...] = e / jnp.sum(e, axis=-1, keepdims=True)
    mu_ref[...] = jnp.dot(g, muw_ref[...],
                          preferred_element_type=_f32) + mub_ref[...]
    sg_ref[...] = jnp.exp(jnp.dot(g, sgw_ref[...],
                                  preferred_element_type=_f32)
                          + sgb_ref[...]) + ms_ref[0, 0]


_fin = pl.pallas_call(
    _fin_body,
    grid=(_GRID,),
    in_specs=[
        pl.BlockSpec((2, _BN, _D), lambda i: (0, i, 0)),
        pl.BlockSpec((_BN, _D), lambda i: (i, 0)),
        pl.BlockSpec((_BN, _D), lambda i: (i, 0)),
        pl.BlockSpec((_BN, 1), lambda i: (i, 0)),
        pl.BlockSpec((1, _D), lambda i: (0, 0)),
        pl.BlockSpec((1, _D), lambda i: (0, 0)),
        pl.BlockSpec((1, _D), lambda i: (0, 0)),
        pl.BlockSpec((1, _D), lambda i: (0, 0)),
        pl.BlockSpec((1, _D), lambda i: (0, 0)),
        pl.BlockSpec((_D, _G), lambda i: (0, 0)),
        pl.BlockSpec((1, _G), lambda i: (0, 0)),
        pl.BlockSpec((_D, _G), lambda i: (0, 0)),
        pl.BlockSpec((1, _G), lambda i: (0, 0)),
        pl.BlockSpec((_D, _G), lambda i: (0, 0)),
        pl.BlockSpec((1, _G), lambda i: (0, 0)),
        pl.BlockSpec((1, 1), lambda i: (0, 0)),
    ],
    out_specs=[
        pl.BlockSpec((_BN, _G), lambda i: (i, 0)),
        pl.BlockSpec((_BN, _G), lambda i: (i, 0)),
        pl.BlockSpec((_BN, _G), lambda i: (i, 0)),
    ],
    out_shape=[
        jax.ShapeDtypeStruct((_NP, _G), _f32),
        jax.ShapeDtypeStruct((_NP, _G), _f32),
        jax.ShapeDtypeStruct((_NP, _G), _f32),
    ],
)


def kernel(x, edge_index, Wg, bg, lng, lnb, g_out, b_out, pi_W, pi_b,
           mu_W, mu_b, sigma_W, sigma_b, min_sigma):
    n = x.shape[0]
    e = edge_index.shape[1]
    src = edge_index[0].astype(jnp.int32)
    dst = edge_index[1].astype(jnp.int32)
    pad_e = jnp.full((_EP - e,), _NP - 1, jnp.int32)
    src2d = jnp.concatenate([src, pad_e]).reshape(_TCH, _CH)
    dst2d = jnp.concatenate([dst, pad_e]).reshape(_TCH, _CH)
    x_p = jnp.pad(x, ((0, _NP - n), (0, 0)))

    deg_parts = _deg_kernel(dst2d)
    dinv, hp = _prep(deg_parts, x_p, Wg[0])

    h = x_p
    for l in range(_L - 1):
        aggp = _agg_kernel(hp, src2d, dst2d)
        h, hp = _mid(aggp, hp, h, dinv,
                     bg[l].reshape(1, _D), lng[l].reshape(1, _D),
                     lnb[l].reshape(1, _D), Wg[l + 1])

    aggp = _agg_kernel(hp, src2d, dst2d)
    l = _L - 1
    pi, mu, sg = _fin(aggp, hp, h, dinv,
                      bg[l].reshape(1, _D), lng[l].reshape(1, _D),
                      lnb[l].reshape(1, _D),
                      g_out.reshape(1, _D), b_out.reshape(1, _D),
                      pi_W, pi_b.reshape(1, _G),
                      mu_W, mu_b.reshape(1, _G),
                      sigma_W, sigma_b.reshape(1, _G),
                      min_sigma.reshape(1, 1))

    pi = pi[:n]
    mu = mu[:n].reshape(n, _G, 1)
    sg = sg[:n].reshape(n, _G, 1)
    return (pi, mu, sg)
